# Initial kernel scaffold; baseline (speedup 1.0000x reference)
import jax
import jax.numpy as jnp
from jax.experimental import pallas as pl


def kernel(atom_in_fea, nbr_fea, self_fea_idx, nbr_fea_idx, W, b, bn1_g, bn1_b, bn2_g, bn2_b):
    raise NotImplementedError("write your pallas kernel here")


# speedup vs baseline: 1.6075x; 1.6075x over previous
import functools

import jax
import jax.numpy as jnp
from jax import lax
from jax.experimental import pallas as pl
from jax.experimental.pallas import tpu as pltpu
from jax.experimental.pallas import tpu_sc as plsc

N_NODES = 10000
N_EDGES = 160000
F = 256
H = 512
NBR = 16
EPS = 1e-5

NC = 2
NS = 16
NW = NC * NS


_A_BLK = 1000


def _proj_body(x_ref, w1_ref, w2_ref, o1_ref, o2_ref):
    x = x_ref[...]
    o1_ref[...] = jnp.dot(x, w1_ref[...], preferred_element_type=jnp.float32)
    o2_ref[...] = jnp.dot(x, w2_ref[...], preferred_element_type=jnp.float32)


def _node_proj(x, w_self, w_nbr):
    grid = N_NODES // _A_BLK
    return pl.pallas_call(
        _proj_body,
        grid=(grid,),
        in_specs=[
            pl.BlockSpec((_A_BLK, F), lambda i: (i, 0)),
            pl.BlockSpec((F, H), lambda i: (0, 0)),
            pl.BlockSpec((F, H), lambda i: (0, 0)),
        ],
        out_specs=[
            pl.BlockSpec((_A_BLK, H), lambda i: (i, 0)),
            pl.BlockSpec((_A_BLK, H), lambda i: (i, 0)),
        ],
        out_shape=[
            jax.ShapeDtypeStruct((N_NODES, H), jnp.float32),
            jax.ShapeDtypeStruct((N_NODES, H), jnp.float32),
        ],
    )(x, w_self, w_nbr)


_B_EPW = N_EDGES // NW
_B_CH = 40
_B_NCH = _B_EPW // _B_CH


def _sc_gather_body(ps_hbm, pn_hbm, si_hbm, ni_hbm, out_hbm,
                    si_v, ni_v, a_v, b_v, sem_a, sem_b):
    wid = lax.axis_index("s") * NC + lax.axis_index("c")
    base = wid * _B_EPW
    pltpu.sync_copy(si_hbm.at[pl.ds(base, _B_EPW)], si_v)
    pltpu.sync_copy(ni_hbm.at[pl.ds(base, _B_EPW)], ni_v)

    def chunk(i, carry):
        e0 = base + i * _B_CH
        cpa = pltpu.async_copy(ps_hbm.at[si_v.at[pl.ds(i * _B_CH, _B_CH)]],
                               a_v, sem_a)
        cpb = pltpu.async_copy(pn_hbm.at[ni_v.at[pl.ds(i * _B_CH, _B_CH)]],
                               b_v, sem_b)
        cpa.wait()
        cpb.wait()

        def row(r, c2):
            for k in range(H // 16):
                sl = pl.ds(k * 16, 16)
                a_v[r, sl] = a_v[r, sl] + b_v[r, sl]
            return c2

        lax.fori_loop(0, _B_CH, row, 0, unroll=False)
        pltpu.sync_copy(a_v, out_hbm.at[pl.ds(e0, _B_CH)])
        return carry

    lax.fori_loop(0, _B_NCH, chunk, 0, unroll=False)


def _sc_gather(p_self, p_nbr, si, ni):
    mesh = plsc.VectorSubcoreMesh(core_axis_name="c", subcore_axis_name="s")
    k = functools.partial(
        pl.kernel,
        out_type=jax.ShapeDtypeStruct((N_EDGES, H), jnp.float32),
        mesh=mesh,
        scratch_types=[
            pltpu.VMEM((_B_EPW,), jnp.int32),
            pltpu.VMEM((_B_EPW,), jnp.int32),
            pltpu.VMEM((_B_CH, H), jnp.float32),
            pltpu.VMEM((_B_CH, H), jnp.float32),
            pltpu.SemaphoreType.DMA,
            pltpu.SemaphoreType.DMA,
        ],
    )(_sc_gather_body)
    return k(p_self, p_nbr, si, ni)


_C_BLK = 2000
_C_GRID = N_EDGES // _C_BLK


def _stats_body(g_ref, nf_ref, we_ref, o_ref):
    t = g_ref[...] + jnp.dot(nf_ref[...], we_ref[...],
                             preferred_element_type=jnp.float32)
    s = jnp.sum(t, axis=0)
    s2 = jnp.sum(t * t, axis=0)

    @pl.when(pl.program_id(0) == 0)
    def _():
        o_ref[...] = jnp.zeros_like(o_ref)

    o_ref[0, :] += s
    o_ref[1, :] += s2


def _edge_stats(g, nbr_fea, w_e):
    return pl.pallas_call(
        _stats_body,
        grid=(_C_GRID,),
        in_specs=[
            pl.BlockSpec((_C_BLK, H), lambda i: (i, 0)),
            pl.BlockSpec((_C_BLK, NBR), lambda i: (i, 0)),
            pl.BlockSpec((NBR, H), lambda i: (0, 0)),
        ],
        out_specs=pl.BlockSpec((2, H), lambda i: (0, 0)),
        out_shape=jax.ShapeDtypeStruct((2, H), jnp.float32),
    )(g, nbr_fea, w_e)


def _softplus(x):
    return jnp.maximum(x, 0.0) + jnp.log1p(jnp.exp(-jnp.abs(x)))


def _gate_body(g_ref, nf_ref, we_ref, sc_ref, sh_ref, o_ref):
    t = g_ref[...] + jnp.dot(nf_ref[...], we_ref[...],
                             preferred_element_type=jnp.float32)
    t = t * sc_ref[...] + sh_ref[...]
    filt = jax.nn.sigmoid(t[:, :F])
    core = _softplus(t[:, F:])
    o_ref[...] = filt * core


def _edge_gate(g, nbr_fea, w_e, scale1, shift1):
    return pl.pallas_call(
        _gate_body,
        grid=(_C_GRID,),
        in_specs=[
            pl.BlockSpec((_C_BLK, H), lambda i: (i, 0)),
            pl.BlockSpec((_C_BLK, NBR), lambda i: (i, 0)),
            pl.BlockSpec((NBR, H), lambda i: (0, 0)),
            pl.BlockSpec((1, H), lambda i: (0, 0)),
            pl.BlockSpec((1, H), lambda i: (0, 0)),
        ],
        out_specs=pl.BlockSpec((_C_BLK, F), lambda i: (i, 0)),
        out_shape=jax.ShapeDtypeStruct((N_EDGES, F), jnp.float32),
    )(g, nbr_fea, w_e, scale1, shift1)


_E_FPC = F // NC
_E_EPT = N_EDGES // NS
_E_CH = 80
_E_NCH = _E_EPT // _E_CH
_E_RCH = 200
_E_NRCH = N_NODES // _E_RCH


def _sc_segsum_body(msg_hbm, si_hbm, out_hbm, acc_sh, idx_v, rows_v, zero_v):
    cid = lax.axis_index("c")
    sid = lax.axis_index("s")

    def zfill(k, c2):
        zero_v[k, pl.ds(0, 16)] = jnp.zeros((16,), jnp.float32)
        for j in range(1, _E_FPC // 16):
            zero_v[k, pl.ds(j * 16, 16)] = jnp.zeros((16,), jnp.float32)
        return c2

    lax.fori_loop(0, _E_RCH, zfill, 0, unroll=False)

    def zrow(t, c2):
        c = sid + t * NS

        @pl.when(c < _E_NRCH)
        def _():
            pltpu.sync_copy(zero_v, acc_sh.at[pl.ds(c * _E_RCH, _E_RCH)])

        return c2

    lax.fori_loop(0, (_E_NRCH + NS - 1) // NS, zrow, 0, unroll=False)
    plsc.subcore_barrier()

    ebase = sid * _E_EPT

    def chunk(i, carry):
        e0 = ebase + i * _E_CH
        pltpu.sync_copy(si_hbm.at[pl.ds(e0, _E_CH)], idx_v)
        pltpu.sync_copy(
            msg_hbm.at[pl.ds(e0, _E_CH), pl.ds(cid * _E_FPC, _E_FPC)],
            rows_v)
        pltpu.sync_copy(rows_v, acc_sh.at[idx_v], add=True)
        return carry

    lax.fori_loop(0, _E_NCH, chunk, 0, unroll=False)
    plsc.subcore_barrier()

    def wrow(t, c2):
        c = sid + t * NS

        @pl.when(c < _E_NRCH)
        def _():
            pltpu.sync_copy(
                acc_sh.at[pl.ds(c * _E_RCH, _E_RCH)],
                out_hbm.at[pl.ds(c * _E_RCH, _E_RCH),
                           pl.ds(cid * _E_FPC, _E_FPC)])

        return c2

    lax.fori_loop(0, (_E_NRCH + NS - 1) // NS, wrow, 0, unroll=False)


def _sc_segsum(msg, si):
    mesh = plsc.VectorSubcoreMesh(core_axis_name="c", subcore_axis_name="s")
    k = functools.partial(
        pl.kernel,
        out_type=jax.ShapeDtypeStruct((N_NODES, F), jnp.float32),
        mesh=mesh,
        scratch_types=[
            pltpu.VMEM_SHARED((N_NODES, _E_FPC), jnp.float32),
            pltpu.VMEM((_E_CH,), jnp.int32),
            pltpu.VMEM((_E_CH, _E_FPC), jnp.float32),
            pltpu.VMEM((_E_RCH, _E_FPC), jnp.float32),
        ],
    )(_sc_segsum_body)
    return k(msg, si)


_F_BLK = 2000
_F_GRID = N_NODES // _F_BLK


def _nstats_body(x_ref, o_ref):
    x = x_ref[...]

    @pl.when(pl.program_id(0) == 0)
    def _():
        o_ref[...] = jnp.zeros_like(o_ref)

    o_ref[0, :] += jnp.sum(x, axis=0)
    o_ref[1, :] += jnp.sum(x * x, axis=0)


def _node_stats(ns):
    return pl.pallas_call(
        _nstats_body,
        grid=(_F_GRID,),
        in_specs=[pl.BlockSpec((_F_BLK, F), lambda i: (i, 0))],
        out_specs=pl.BlockSpec((2, F), lambda i: (0, 0)),
        out_shape=jax.ShapeDtypeStruct((2, F), jnp.float32),
    )(ns)


def _out_body(x_ref, ns_ref, sc_ref, sh_ref, o_ref):
    v = x_ref[...] + ns_ref[...] * sc_ref[...] + sh_ref[...]
    o_ref[...] = _softplus(v)


def _node_out(atom, ns, scale2, shift2):
    return pl.pallas_call(
        _out_body,
        grid=(_F_GRID,),
        in_specs=[
            pl.BlockSpec((_F_BLK, F), lambda i: (i, 0)),
            pl.BlockSpec((_F_BLK, F), lambda i: (i, 0)),
            pl.BlockSpec((1, F), lambda i: (0, 0)),
            pl.BlockSpec((1, F), lambda i: (0, 0)),
        ],
        out_specs=pl.BlockSpec((_F_BLK, F), lambda i: (i, 0)),
        out_shape=jax.ShapeDtypeStruct((N_NODES, F), jnp.float32),
    )(atom, ns, scale2, shift2)


def kernel(atom_in_fea, nbr_fea, self_fea_idx, nbr_fea_idx, W, b,
           bn1_g, bn1_b, bn2_g, bn2_b):
    del b
    si = self_fea_idx.astype(jnp.int32)
    ni = nbr_fea_idx.astype(jnp.int32)
    w_self = W[:F, :]
    w_nbr = W[F:2 * F, :]
    w_e = W[2 * F:, :]

    p_self, p_nbr = _node_proj(atom_in_fea, w_self, w_nbr)
    g = _sc_gather(p_self, p_nbr, si, ni)

    st1 = _edge_stats(g, nbr_fea, w_e)
    mu1 = st1[0] / N_EDGES
    var1 = st1[1] / N_EDGES - mu1 * mu1
    scale1 = bn1_g * lax.rsqrt(var1 + EPS)
    shift1 = bn1_b - mu1 * scale1

    msg = _edge_gate(g, nbr_fea, w_e, scale1[None, :], shift1[None, :])
    nbr_sumed = _sc_segsum(msg, si)

    st2 = _node_stats(nbr_sumed)
    mu2 = st2[0] / N_NODES
    var2 = st2[1] / N_NODES - mu2 * mu2
    scale2 = bn2_g * lax.rsqrt(var2 + EPS)
    shift2 = bn2_b - mu2 * scale2

    return _node_out(atom_in_fea, nbr_sumed, scale2[None, :], shift2[None, :])
